# baseline (speedup 1.0000x reference)
import functools

import jax
import jax.numpy as jnp
from jax import lax
from jax.experimental import pallas as pl
from jax.experimental.pallas import tpu as pltpu
from jax.experimental.pallas import tpu_sc as plsc

VOCAB = 1_000_000
D = 64
L_CTX = 200
B = 1024
N_ROWS = B * L_CTX
NC, NS = 2, 16
NW = NC * NS
ROWS_PER_W = N_ROWS // NW
G_SUB = 100
CHUNK = 400
N_SUB = CHUNK // G_SUB
N_CHUNKS = ROWS_PER_W // CHUNK
LANES = 16


def _body(idx_hbm, tok_hbm, pos_hbm, out_hbm, idx_v, pos_v, rows_v, gsem):
    wid = lax.axis_index("s") * NC + lax.axis_index("c")
    base = wid * ROWS_PER_W

    pltpu.sync_copy(idx_hbm.at[wid], idx_v)
    pltpu.sync_copy(pos_hbm, pos_v)

    def chunk_step(c, carry):
        copies = []
        for j in range(N_SUB):
            copies.append(
                pltpu.async_copy(
                    tok_hbm.at[idx_v.at[c * N_SUB + j]],
                    rows_v.at[pl.ds(j * G_SUB, G_SUB)],
                    gsem,
                )
            )
        for cp in copies:
            cp.wait()

        def add_step(r, carry2):
            for s in range(D // LANES):
                sl = pl.ds(s * LANES, LANES)
                pv = pos_v[r, sl]
                rows_v[r, sl] += pv
                rows_v[r + L_CTX, sl] += pv
            return carry2

        lax.fori_loop(0, L_CTX, add_step, 0, unroll=2)

        pltpu.sync_copy(
            rows_v, out_hbm.at[pl.ds(base + c * CHUNK, CHUNK)]
        )
        return carry

    lax.fori_loop(0, N_CHUNKS, chunk_step, 0)


@functools.partial(jax.jit, static_argnames=())
def _embed(idx, tok, pos):
    mesh = plsc.VectorSubcoreMesh(
        core_axis_name="c", subcore_axis_name="s", num_cores=NC, num_subcores=NS
    )
    f = pl.kernel(
        _body,
        out_type=jax.ShapeDtypeStruct((N_ROWS, D), jnp.float32),
        mesh=mesh,
        scratch_types=[
            pltpu.VMEM((ROWS_PER_W // G_SUB, G_SUB), jnp.int32),
            pltpu.VMEM((L_CTX, D), jnp.float32),
            pltpu.VMEM((CHUNK, D), jnp.float32),
            pltpu.SemaphoreType.DMA,
        ],
        compiler_params=pltpu.CompilerParams(use_tc_tiling_on_sc=False),
    )
    return f(idx, tok, pos)


def kernel(inputs, token_table, position_table):
    idx = inputs.astype(jnp.int32).reshape(NW, ROWS_PER_W // G_SUB, G_SUB)
    out = _embed(idx, token_table, position_table)
    return out.reshape(B, L_CTX, D)


# speedup vs baseline: 1.0474x; 1.0474x over previous
import functools

import jax
import jax.numpy as jnp
from jax import lax
from jax.experimental import pallas as pl
from jax.experimental.pallas import tpu as pltpu
from jax.experimental.pallas import tpu_sc as plsc

VOCAB = 1_000_000
D = 64
L_CTX = 200
B = 1024
N_ROWS = B * L_CTX
NC, NS = 2, 16
NW = NC * NS
ROWS_PER_W = N_ROWS // NW
G_SUB = 100
CHUNK = 400
N_SUB = CHUNK // G_SUB
N_CHUNKS = ROWS_PER_W // CHUNK
NBUF = 3
LANES = 16


def _body(idx_hbm, tok_hbm, pos_hbm, out_hbm,
          idx_v, pos_v, rows0, rows1, rows2,
          gsem0, gsem1, gsem2, ssem0, ssem1, ssem2):
    rows = (rows0, rows1, rows2)
    gsems = (gsem0, gsem1, gsem2)
    ssems = (ssem0, ssem1, ssem2)

    wid = lax.axis_index("s") * NC + lax.axis_index("c")
    base = wid * ROWS_PER_W

    pltpu.sync_copy(idx_hbm.at[wid], idx_v)
    pltpu.sync_copy(pos_hbm, pos_v)

    def fire_gather(c):
        b = c % NBUF
        return [
            pltpu.async_copy(
                tok_hbm.at[idx_v.at[c * N_SUB + j]],
                rows[b].at[pl.ds(j * G_SUB, G_SUB)],
                gsems[b],
            )
            for j in range(N_SUB)
        ]

    def compute(b):
        rv = rows[b]

        def add_step(r, carry):
            for s in range(D // LANES):
                sl = pl.ds(s * LANES, LANES)
                pv = pos_v[r, sl]
                rv[r, sl] += pv
                rv[r + L_CTX, sl] += pv
            return carry

        lax.fori_loop(0, L_CTX, add_step, 0, unroll=4)

    gdesc = {0: fire_gather(0), 1: fire_gather(1)}
    sdesc = {}
    for c in range(N_CHUNKS):
        b = c % NBUF
        for cp in gdesc.pop(b):
            cp.wait()
        compute(b)
        sdesc[b] = pltpu.async_copy(
            rows[b], out_hbm.at[pl.ds(base + c * CHUNK, CHUNK)], ssems[b]
        )
        if c >= 1:
            bp = (c - 1) % NBUF
            sdesc.pop(bp).wait()
        if c + 2 < N_CHUNKS:
            gdesc[(c + 2) % NBUF] = fire_gather(c + 2)
    for cp in sdesc.values():
        cp.wait()


@functools.partial(jax.jit, static_argnames=())
def _embed(idx, tok, pos):
    mesh = plsc.VectorSubcoreMesh(
        core_axis_name="c", subcore_axis_name="s", num_cores=NC, num_subcores=NS
    )
    f = pl.kernel(
        _body,
        out_type=jax.ShapeDtypeStruct((N_ROWS, D), jnp.float32),
        mesh=mesh,
        scratch_types=[
            pltpu.VMEM((ROWS_PER_W // G_SUB, G_SUB), jnp.int32),
            pltpu.VMEM((L_CTX, D), jnp.float32),
            pltpu.VMEM((CHUNK, D), jnp.float32),
            pltpu.VMEM((CHUNK, D), jnp.float32),
            pltpu.VMEM((CHUNK, D), jnp.float32),
            pltpu.SemaphoreType.DMA,
            pltpu.SemaphoreType.DMA,
            pltpu.SemaphoreType.DMA,
            pltpu.SemaphoreType.DMA,
            pltpu.SemaphoreType.DMA,
            pltpu.SemaphoreType.DMA,
        ],
        compiler_params=pltpu.CompilerParams(use_tc_tiling_on_sc=False),
    )
    return f(idx, tok, pos)


def kernel(inputs, token_table, position_table):
    idx = inputs.astype(jnp.int32).reshape(NW, ROWS_PER_W // G_SUB, G_SUB)
    out = _embed(idx, token_table, position_table)
    return out.reshape(B, L_CTX, D)
